# Initial kernel scaffold; baseline (speedup 1.0000x reference)
import jax
import jax.numpy as jnp
from jax.experimental import pallas as pl


def kernel(x, coords, Wq, Wk, Wv):
    raise NotImplementedError("write your pallas kernel here")


# speedup vs baseline: 70.6447x; 70.6447x over previous
import functools

import jax
import jax.numpy as jnp
from jax import lax
from jax.experimental import pallas as pl

_B, _S, _D, _H, _K = 2, 2048, 768, 12, 16
_HD = _D // _H
_SCALE = _HD ** (-0.5)

_RB = 256
_QB = 256
_BIG = 3.0e38
_NEG = -1.0e30


def _proj_body(x_ref, wq_ref, wk_ref, wv_ref, q_ref, k_ref, v_ref, m_ref):
    x = x_ref[0].astype(jnp.bfloat16)
    wq = wq_ref[...].astype(jnp.bfloat16)
    wk = wk_ref[...].astype(jnp.bfloat16)
    wv = wv_ref[...].astype(jnp.bfloat16)
    dn = (((1,), (1,)), ((), ()))
    q = lax.dot_general(x, wq, dn, preferred_element_type=jnp.float32)
    k = lax.dot_general(x, wk, dn, preferred_element_type=jnp.float32)
    v = lax.dot_general(x, wv, dn, preferred_element_type=jnp.float32)
    q_ref[0] = q
    k_ref[0] = k
    v_ref[0] = v
    acc = jnp.zeros((x.shape[0], _HD), jnp.float32)
    for h in range(_H):
        acc = acc + k[:, h * _HD:(h + 1) * _HD]
    m_ref[0] = acc * (1.0 / _H)


def _attn_body(q_ref, k_ref, v_ref, cq_ref, ck_ref, out_ref):
    jq = pl.program_id(1)
    cq = cq_ref[0]
    ckt = ck_ref[0]
    qn = (cq[:, 0:1] * cq[:, 0:1] + cq[:, 1:2] * cq[:, 1:2]
          + cq[:, 2:3] * cq[:, 2:3])
    kn = (ckt[0:1, :] * ckt[0:1, :] + ckt[1:2, :] * ckt[1:2, :]
          + ckt[2:3, :] * ckt[2:3, :])
    cross = lax.dot_general(cq.astype(jnp.bfloat16), ckt.astype(jnp.bfloat16),
                            (((1,), (0,)), ((), ())),
                            preferred_element_type=jnp.float32)
    d = (qn + kn) - 2.0 * cross
    rid = jq * _QB + lax.broadcasted_iota(jnp.int32, (_QB, _S), 0)
    cid = lax.broadcasted_iota(jnp.int32, (_QB, _S), 1)
    d = jnp.where(rid == cid, _BIG, d)

    def body(i, carry):
        dcur, _ = carry
        m = jnp.min(dcur, axis=1, keepdims=True)
        dcur = jnp.where(dcur <= m, _BIG, dcur)
        return (dcur, m)

    _, thr = lax.fori_loop(0, _K, body, (d, jnp.zeros((_QB, 1), jnp.float32)))
    nmask = d <= thr

    dn_t = (((1,), (1,)), ((), ()))
    dn_n = (((1,), (0,)), ((), ()))
    for h in range(_H):
        sl = slice(h * _HD, (h + 1) * _HD)
        qh = q_ref[0, :, sl].astype(jnp.bfloat16)
        kh = k_ref[0, :, sl].astype(jnp.bfloat16)
        vh = v_ref[0, :, sl].astype(jnp.bfloat16)
        s = lax.dot_general(qh, kh, dn_t,
                            preferred_element_type=jnp.float32) * _SCALE
        s = jnp.where(nmask, s, _NEG)
        m = jnp.max(s, axis=1, keepdims=True)
        p = jnp.exp(s - m)
        denom = jnp.sum(p, axis=1, keepdims=True)
        p = (p / denom).astype(jnp.bfloat16)
        oh = lax.dot_general(p, vh, dn_n, preferred_element_type=jnp.float32)
        out_ref[0, :, sl] = oh


@jax.jit
def kernel(x, coords, Wq, Wk, Wv):
    q, k, v, metric = pl.pallas_call(
        _proj_body,
        grid=(_B, _S // _RB),
        in_specs=[
            pl.BlockSpec((1, _RB, _D), lambda b, j: (b, j, 0)),
            pl.BlockSpec((_D, _D), lambda b, j: (0, 0)),
            pl.BlockSpec((_D, _D), lambda b, j: (0, 0)),
            pl.BlockSpec((_D, _D), lambda b, j: (0, 0)),
        ],
        out_specs=[
            pl.BlockSpec((1, _RB, _D), lambda b, j: (b, j, 0)),
            pl.BlockSpec((1, _RB, _D), lambda b, j: (b, j, 0)),
            pl.BlockSpec((1, _RB, _D), lambda b, j: (b, j, 0)),
            pl.BlockSpec((1, _RB, _HD), lambda b, j: (b, j, 0)),
        ],
        out_shape=[
            jax.ShapeDtypeStruct((_B, _S, _D), jnp.float32),
            jax.ShapeDtypeStruct((_B, _S, _D), jnp.float32),
            jax.ShapeDtypeStruct((_B, _S, _D), jnp.float32),
            jax.ShapeDtypeStruct((_B, _S, _HD), jnp.float32),
        ],
    )(x, Wq, Wk, Wv)

    coords_pad = jnp.pad(coords, ((0, 0), (0, 0), (0, 5)))
    coords_t = jnp.pad(jnp.swapaxes(coords, 1, 2),
                       ((0, 0), (0, 5), (0, 0)))

    out = pl.pallas_call(
        _attn_body,
        grid=(_B, _S // _QB),
        in_specs=[
            pl.BlockSpec((1, _QB, _D), lambda b, j: (b, j, 0)),
            pl.BlockSpec((1, _S, _D), lambda b, j: (b, 0, 0)),
            pl.BlockSpec((1, _S, _D), lambda b, j: (b, 0, 0)),
            pl.BlockSpec((1, _QB, 8), lambda b, j: (b, j, 0)),
            pl.BlockSpec((1, 8, _S), lambda b, j: (b, 0, 0)),
        ],
        out_specs=pl.BlockSpec((1, _QB, _D), lambda b, j: (b, j, 0)),
        out_shape=jax.ShapeDtypeStruct((_B, _S, _D), jnp.float32),
    )(q, k, v, coords_pad, coords_t)

    return (out, metric)


# speedup vs baseline: 93.6411x; 1.3255x over previous
import jax
import jax.numpy as jnp
from jax import lax
from jax.experimental import pallas as pl

_B, _S, _D, _H, _K = 2, 2048, 768, 12, 16
_HD = _D // _H
_SCALE = _HD ** (-0.5)

_RB = 256
_QB = 256
_BIG = 3.0e38
_NEG = -1.0e30


def _proj_body(x_ref, wq_ref, wk_ref, wv_ref, q_ref, k_ref, v_ref, m_ref):
    x = x_ref[0].astype(jnp.bfloat16)
    wq = wq_ref[...].astype(jnp.bfloat16)
    wk = wk_ref[...].astype(jnp.bfloat16)
    wv = wv_ref[...].astype(jnp.bfloat16)
    dn = (((1,), (1,)), ((), ()))
    q = lax.dot_general(x, wq, dn, preferred_element_type=jnp.float32)
    k = lax.dot_general(x, wk, dn, preferred_element_type=jnp.float32)
    v = lax.dot_general(x, wv, dn, preferred_element_type=jnp.float32)
    q_ref[0] = (q * _SCALE).astype(jnp.bfloat16)
    k_ref[0] = k.astype(jnp.bfloat16)
    v_ref[0] = v.astype(jnp.bfloat16)
    acc = jnp.zeros((x.shape[0], _HD), jnp.float32)
    for h in range(_H):
        acc = acc + k[:, h * _HD:(h + 1) * _HD]
    m_ref[0] = acc * (1.0 / _H)


def _attn_body(q_ref, k_ref, v_ref, cq_ref, ck_ref, out_ref):
    jq = pl.program_id(1)
    cq = cq_ref[0]
    ckt = ck_ref[0]
    qn = (cq[:, 0:1] * cq[:, 0:1] + cq[:, 1:2] * cq[:, 1:2]
          + cq[:, 2:3] * cq[:, 2:3])
    kn = (ckt[0:1, :] * ckt[0:1, :] + ckt[1:2, :] * ckt[1:2, :]
          + ckt[2:3, :] * ckt[2:3, :])
    cross = lax.dot_general(cq.astype(jnp.bfloat16), ckt.astype(jnp.bfloat16),
                            (((1,), (0,)), ((), ())),
                            preferred_element_type=jnp.float32)
    d = (qn + kn) - 2.0 * cross
    rid = jq * _QB + lax.broadcasted_iota(jnp.int32, (_QB, _S), 0)
    cid = lax.broadcasted_iota(jnp.int32, (_QB, _S), 1)
    d = jnp.where(rid == cid, _BIG, d)

    def body(i, carry):
        dcur, _ = carry
        m = jnp.min(dcur, axis=1, keepdims=True)
        dcur = jnp.where(dcur <= m, _BIG, dcur)
        return (dcur, m)

    _, thr = lax.fori_loop(0, _K, body, (d, jnp.zeros((_QB, 1), jnp.float32)))
    bias = jnp.where(d <= thr, 0.0, _NEG)

    dn_t = (((1,), (1,)), ((), ()))
    dn_n = (((1,), (0,)), ((), ()))
    for h in range(_H):
        sl = slice(h * _HD, (h + 1) * _HD)
        qh = q_ref[0, :, sl]
        kh = k_ref[0, :, sl]
        vh = v_ref[0, :, sl]
        s = lax.dot_general(qh, kh, dn_t, preferred_element_type=jnp.float32)
        p = jnp.exp(s + bias)
        denom = jnp.sum(p, axis=1, keepdims=True)
        oh = lax.dot_general(p.astype(jnp.bfloat16), vh, dn_n,
                             preferred_element_type=jnp.float32)
        out_ref[0, :, sl] = oh / denom


@jax.jit
def kernel(x, coords, Wq, Wk, Wv):
    q, k, v, metric = pl.pallas_call(
        _proj_body,
        grid=(_B, _S // _RB),
        in_specs=[
            pl.BlockSpec((1, _RB, _D), lambda b, j: (b, j, 0)),
            pl.BlockSpec((_D, _D), lambda b, j: (0, 0)),
            pl.BlockSpec((_D, _D), lambda b, j: (0, 0)),
            pl.BlockSpec((_D, _D), lambda b, j: (0, 0)),
        ],
        out_specs=[
            pl.BlockSpec((1, _RB, _D), lambda b, j: (b, j, 0)),
            pl.BlockSpec((1, _RB, _D), lambda b, j: (b, j, 0)),
            pl.BlockSpec((1, _RB, _D), lambda b, j: (b, j, 0)),
            pl.BlockSpec((1, _RB, _HD), lambda b, j: (b, j, 0)),
        ],
        out_shape=[
            jax.ShapeDtypeStruct((_B, _S, _D), jnp.bfloat16),
            jax.ShapeDtypeStruct((_B, _S, _D), jnp.bfloat16),
            jax.ShapeDtypeStruct((_B, _S, _D), jnp.bfloat16),
            jax.ShapeDtypeStruct((_B, _S, _HD), jnp.float32),
        ],
    )(x, Wq, Wk, Wv)

    coords_pad = jnp.pad(coords, ((0, 0), (0, 0), (0, 5)))
    coords_t = jnp.pad(jnp.swapaxes(coords, 1, 2),
                       ((0, 0), (0, 5), (0, 0)))

    out = pl.pallas_call(
        _attn_body,
        grid=(_B, _S // _QB),
        in_specs=[
            pl.BlockSpec((1, _QB, _D), lambda b, j: (b, j, 0)),
            pl.BlockSpec((1, _S, _D), lambda b, j: (b, 0, 0)),
            pl.BlockSpec((1, _S, _D), lambda b, j: (b, 0, 0)),
            pl.BlockSpec((1, _QB, 8), lambda b, j: (b, j, 0)),
            pl.BlockSpec((1, 8, _S), lambda b, j: (b, 0, 0)),
        ],
        out_specs=pl.BlockSpec((1, _QB, _D), lambda b, j: (b, j, 0)),
        out_shape=jax.ShapeDtypeStruct((_B, _S, _D), jnp.float32),
    )(q, k, v, coords_pad, coords_t)

    return (out, metric)
